# baseline (speedup 1.0000x reference)
import functools

import jax
import jax.numpy as jnp
from jax import lax
from jax.experimental import pallas as pl
from jax.experimental.pallas import tpu as pltpu
from jax.experimental.pallas import tpu_sc as plsc

_NC = 2
_NS = 16
_NW = _NC * _NS
_L = 16


@functools.partial(jax.jit, static_argnames=("chunk",))
def _sc_embed(x, table, pe2d, *, chunk):
    b, s = x.shape
    d = table.shape[1]
    s_per_w = s // _NW
    n_chunks = s_per_w // chunk
    x_flat = x.reshape(b * s)
    mesh = plsc.VectorSubcoreMesh(core_axis_name="c", subcore_axis_name="s")

    @functools.partial(
        pl.kernel,
        mesh=mesh,
        out_type=jax.ShapeDtypeStruct((b * s, d), jnp.float32),
        scratch_types=[
            pltpu.VMEM((chunk,), jnp.int32),
            pltpu.VMEM((chunk, d), jnp.float32),
            pltpu.VMEM((chunk, d), jnp.float32),
            pltpu.SemaphoreType.DMA,
        ],
    )
    def k(x_hbm, tab_hbm, pe_hbm, out_hbm, idx_v, pe_v, rows_v, sem):
        wid = lax.axis_index("s") * _NC + lax.axis_index("c")
        s_base = wid * s_per_w

        for c in range(n_chunks):
            s_off = s_base + c * chunk
            pltpu.sync_copy(pe_hbm.at[pl.ds(s_off, chunk)], pe_v)
            for bi in range(b):
                off = bi * s + s_off
                pltpu.sync_copy(x_hbm.at[pl.ds(off, chunk)], idx_v)
                pltpu.async_copy(tab_hbm.at[idx_v], rows_v, sem).wait()

                def add_row(r, carry):
                    for j in range(d // _L):
                        sl = pl.ds(j * _L, _L)
                        plsc.addupdate(rows_v.at[r, sl], pe_v[r, sl])
                    return carry

                lax.fori_loop(0, chunk, add_row, 0)
                pltpu.sync_copy(rows_v, out_hbm.at[pl.ds(off, chunk)])

    out = k(x_flat, table, pe2d)
    return out.reshape(b, s, d)


def kernel(x, table, pe):
    b, s = x.shape
    d = table.shape[1]
    pe2d = pe.reshape(pe.shape[1], d)
    out = _sc_embed(x.astype(jnp.int32), table, pe2d, chunk=64)
    return out


# speedup vs baseline: 1.2632x; 1.2632x over previous
import functools

import jax
import jax.numpy as jnp
from jax import lax
from jax.experimental import pallas as pl
from jax.experimental.pallas import tpu as pltpu
from jax.experimental.pallas import tpu_sc as plsc

_NC = 2
_NS = 16
_NW = _NC * _NS
_L = 16


@functools.partial(jax.jit, static_argnames=("chunk",))
def _sc_embed(x, table, pe2d, *, chunk):
    b, s = x.shape
    d = table.shape[1]
    s_per_w = s // _NW
    n_chunks = s_per_w // chunk
    n_iter = n_chunks * b
    x_flat = x.reshape(b * s)
    mesh = plsc.VectorSubcoreMesh(core_axis_name="c", subcore_axis_name="s")

    @functools.partial(
        pl.kernel,
        mesh=mesh,
        out_type=jax.ShapeDtypeStruct((b * s, d), jnp.float32),
        scratch_types=[
            pltpu.VMEM((b, s_per_w), jnp.int32),
            pltpu.VMEM((chunk, d), jnp.float32),
            pltpu.VMEM((chunk, d), jnp.float32),
            pltpu.VMEM((chunk, d), jnp.float32),
            pltpu.VMEM((chunk, d), jnp.float32),
            pltpu.SemaphoreType.DMA,
            pltpu.SemaphoreType.DMA,
            pltpu.SemaphoreType.DMA,
            pltpu.SemaphoreType.DMA,
            pltpu.SemaphoreType.DMA,
        ],
    )
    def k(x_hbm, tab_hbm, pe_hbm, out_hbm,
          idx_bb, rows0, rows1, pe0, pe1, g0, g1, o0, o1, psem):
        rows = (rows0, rows1)
        pes = (pe0, pe1)
        gsems = (g0, g1)
        osems = (o0, o1)

        wid = lax.axis_index("s") * _NC + lax.axis_index("c")
        s_base = wid * s_per_w

        for bi in range(b):
            pltpu.sync_copy(x_hbm.at[pl.ds(bi * s + s_base, s_per_w)],
                            idx_bb.at[bi])

        def idx_view(c, bi):
            return idx_bb.at[bi, pl.ds(c * chunk, chunk)]

        def out_view(c, bi):
            return out_hbm.at[pl.ds(bi * s + s_base + c * chunk, chunk)]

        pltpu.sync_copy(pe_hbm.at[pl.ds(s_base, chunk)], pe0)
        gdesc = [None, None]
        odesc = [None, None]
        pedesc = None
        gdesc[0] = pltpu.async_copy(tab_hbm.at[idx_view(0, 0)], rows0, g0)

        for i in range(n_iter):
            p = i % 2
            c, bi = divmod(i, b)
            if bi == 0 and c + 1 < n_chunks:
                pedesc = pltpu.async_copy(
                    pe_hbm.at[pl.ds(s_base + (c + 1) * chunk, chunk)],
                    pes[(c + 1) % 2], psem)
            if bi == 0 and c > 0:
                pedesc.wait()
            gdesc[p].wait()
            if i + 1 < n_iter:
                if i >= 1 and odesc[1 - p] is not None:
                    odesc[1 - p].wait()
                cn, bn = divmod(i + 1, b)
                gdesc[1 - p] = pltpu.async_copy(
                    tab_hbm.at[idx_view(cn, bn)], rows[1 - p], gsems[1 - p])

            pe_buf = pes[c % 2]
            rows_buf = rows[p]

            def add_row(r, carry):
                for j in range(d // _L):
                    sl = pl.ds(j * _L, _L)
                    plsc.addupdate(rows_buf.at[r, sl], pe_buf[r, sl])
                return carry

            lax.fori_loop(0, chunk, add_row, 0)
            odesc[p] = pltpu.async_copy(rows_buf, out_view(c, bi), osems[p])

        odesc[0].wait()
        odesc[1].wait()

    out = k(x_flat, table, pe2d)
    return out.reshape(b, s, d)


def kernel(x, table, pe):
    b, s = x.shape
    d = table.shape[1]
    pe2d = pe.reshape(pe.shape[1], d)
    out = _sc_embed(x.astype(jnp.int32), table, pe2d, chunk=32)
    return out
